# baseline (speedup 1.0000x reference)
import functools

import jax
import jax.numpy as jnp
from jax import lax
from jax.experimental import pallas as pl
from jax.experimental.pallas import tpu as pltpu
from jax.experimental.pallas import tpu_sc as plsc

NC = 2
NS = 16
NW = NC * NS

F32 = jnp.float32
HID = 128
BLK = 2048


def _leaky(x):
    return jnp.where(x >= 0, x, 0.2 * x)


def _dot(a, b):
    return jax.lax.dot_general(
        a, b, (((1,), (0,)), ((), ())), preferred_element_type=F32)


def _bf(v):
    return v.astype(jnp.bfloat16).astype(F32)


def _ln(h, g, be):
    mu = jnp.mean(h, axis=-1, keepdims=True)
    d = h - mu
    var = jnp.mean(d * d, axis=-1, keepdims=True)
    return d * jax.lax.rsqrt(var + 1e-5) * g + be


def _full(shape):
    return pl.BlockSpec(shape, lambda i: tuple(0 for _ in shape))


def _rows(last=HID):
    return pl.BlockSpec((BLK, last), lambda i: (i, 0))


def _k2_body(deg_ref, berow_ref, eW1, eb1, eW2, eb2, eW3, eg, ebe,
             edW1, edb1, edW2, edb2, edW3, edg, edbe,
             nW1, nb1, nW2, nb2, nW3, ng, nbe,
             e2W1, e2b1,
             h1_ref, a_ref, b_ref):
    h = _leaky(_dot(berow_ref[...], eW1[...]) + eb1[...])
    h = _leaky(_dot(h, eW2[...]) + eb2[...])
    h0 = _ln(_dot(h, eW3[...]), eg[...], ebe[...])
    t = _leaky(_dot(h0, edW1[0]) + _dot(h0, edW1[1]) + edb1[...])
    t = _leaky(_dot(t, edW2[...]) + edb2[...])
    m = _ln(_dot(t, edW3[...]), edg[...], edbe[...])
    c0 = _dot(h0, nW1[0]) + nb1[...]
    z2 = deg_ref[...] * m
    z = _leaky(_dot(z2, nW1[1]) + c0)
    z = _leaky(_dot(z, nW2[...]) + nb2[...])
    h1 = _ln(_dot(z, nW3[...]), ng[...], nbe[...])
    h1_ref[...] = h1
    a_ref[...] = _dot(h1, e2W1[0]) + e2b1[...]
    b_ref[...] = _dot(h1, e2W1[1])


def _run_k2(deg_col, berow, p, np_, grid):
    enc = p['enc_exp']
    ed0 = p['proc'][0]['edge']
    nd0 = p['proc'][0]['node']
    ed1 = p['proc'][1]['edge']
    r2 = lambda v: v.reshape(1, -1)
    args = [
        deg_col, berow,
        enc['W1'], r2(enc['b1']), enc['W2'], r2(enc['b2']), enc['W3'],
        r2(enc['g']), r2(enc['be']),
        ed0['W1'].reshape(2, HID, HID), r2(ed0['b1']), ed0['W2'],
        r2(ed0['b2']), ed0['W3'], r2(ed0['g']), r2(ed0['be']),
        nd0['W1'].reshape(2, HID, HID), r2(nd0['b1']), nd0['W2'],
        r2(nd0['b2']), nd0['W3'], r2(nd0['g']), r2(nd0['be']),
        ed1['W1'].reshape(2, HID, HID), r2(ed1['b1']),
    ]
    specs = [_rows(1)] + [_full(a.shape) for a in args[1:]]
    out = pl.pallas_call(
        _k2_body,
        grid=(grid,),
        in_specs=specs,
        out_specs=[_rows(), _rows(), _rows()],
        out_shape=[jax.ShapeDtypeStruct((np_, HID), F32)] * 3,
    )(*args)
    return out


def _sc_gather(A, B, desti, srci, ne_pad):
    per_w = ne_pad // NW
    C = 256
    n_chunks = per_w // C
    mesh = plsc.VectorSubcoreMesh(
        core_axis_name="c", subcore_axis_name="s",
        num_cores=NC, num_subcores=NS)

    @functools.partial(
        pl.kernel,
        out_type=jax.ShapeDtypeStruct((ne_pad, HID), F32),
        mesh=mesh,
        scratch_types=[
            pltpu.VMEM((C,), jnp.int32),
            pltpu.VMEM((C,), jnp.int32),
            pltpu.VMEM((C, HID), F32),
            pltpu.SemaphoreType.DMA,
        ],
    )
    def k(dest_hbm, src_hbm, a_hbm, b_hbm, e_hbm, idxd, idxs, rows, sem):
        wid = lax.axis_index("s") * NC + lax.axis_index("c")
        base = wid * per_w

        def body(g, carry):
            off = base + g * C
            pltpu.sync_copy(dest_hbm.at[pl.ds(off, C)], idxd)
            pltpu.sync_copy(src_hbm.at[pl.ds(off, C)], idxs)
            pltpu.async_copy(a_hbm.at[idxd], rows, sem).wait()
            pltpu.async_copy(b_hbm.at[idxs], rows, sem, add=True).wait()
            pltpu.sync_copy(rows, e_hbm.at[pl.ds(off, C)])
            return carry

        lax.fori_loop(0, n_chunks, body, 0)

    return k(desti, srci, A, B)


def _sc_scatter_rows(V, idx, M, NB, P, R):
    S = M // NS
    RT = R + 16
    SCAN = 512
    RZ = R // NS
    assert NB == NC * P * R and S % SCAN == 0 and M % NS == 0
    assert RZ % 16 == 0
    mesh = plsc.VectorSubcoreMesh(
        core_axis_name="c", subcore_axis_name="s",
        num_cores=NC, num_subcores=NS)

    @functools.partial(
        pl.kernel,
        out_type=jax.ShapeDtypeStruct((NB, HID), F32),
        mesh=mesh,
        scratch_types=[
            pltpu.VMEM((SCAN,), jnp.int32),
            pltpu.VMEM((SCAN,), jnp.int32),
            pltpu.VMEM((SCAN, HID), F32),
            pltpu.VMEM((16, HID), F32),
            pltpu.VMEM_SHARED((RT, HID), F32),
            pltpu.SemaphoreType.DMA,
        ],
    )
    def k(v_hbm, idx_hbm, acc_hbm, idxc, lidx_c, rows, zbuf, spm, sem):
        cid = lax.axis_index("c")
        tid = lax.axis_index("s")
        tbase = tid * S
        for i in range(16):
            for j in range(HID // 16):
                zbuf[i, pl.ds(j * 16, 16)] = jnp.zeros((16,), F32)
        trash = jnp.full((16,), R, jnp.int32)
        for p in range(P):
            base = cid * (P * R) + p * R
            for zc in range(RZ // 16):
                pltpu.sync_copy(zbuf, spm.at[pl.ds(tid * RZ + zc * 16, 16)])
            plsc.subcore_barrier()

            def chunk(w, carry):
                cbase = tbase + w * SCAN
                pltpu.sync_copy(idx_hbm.at[pl.ds(cbase, SCAN)], idxc)
                cp = pltpu.async_copy(v_hbm.at[pl.ds(cbase, SCAN)], rows,
                                      sem)

                def win(i, c):
                    v16 = idxc[pl.ds(i * 16, 16)]
                    mask = (v16 >= base) & (v16 < base + R)
                    lidx_c[pl.ds(i * 16, 16)] = jnp.where(
                        mask, v16 - base, trash)
                    return c

                lax.fori_loop(0, SCAN // 16, win, 0)
                cp.wait()
                pltpu.sync_copy(rows, spm.at[lidx_c], add=True)
                return carry

            lax.fori_loop(0, S // SCAN, chunk, 0)
            plsc.subcore_barrier()
            rbase = tid * RZ
            nfull, rem = divmod(RZ, 256)
            wchunks = [(i * 256, 256) for i in range(nfull)]
            if rem:
                wchunks.append((nfull * 256, rem))
            for oc, ln in wchunks:
                pltpu.sync_copy(spm.at[pl.ds(rbase + oc, ln)],
                                rows.at[pl.ds(0, ln)])
                pltpu.sync_copy(
                    rows.at[pl.ds(0, ln)],
                    acc_hbm.at[pl.ds(base + rbase + oc, ln)])
            plsc.subcore_barrier()

    return k(V, idx)


def _sc_degree(idx, M, NB, P, R):
    S = M // NS
    SCAN = 512
    RT = R + 16
    RZ = R // NS
    W = 16
    assert NB == NC * P * R and S % SCAN == 0 and RZ % 16 == 0
    mesh = plsc.VectorSubcoreMesh(
        core_axis_name="c", subcore_axis_name="s",
        num_cores=NC, num_subcores=NS)

    @functools.partial(
        pl.kernel,
        out_type=jax.ShapeDtypeStruct((NB, W), F32),
        mesh=mesh,
        scratch_types=[
            pltpu.VMEM((SCAN,), jnp.int32),
            pltpu.VMEM((SCAN,), jnp.int32),
            pltpu.VMEM((SCAN, W), F32),
            pltpu.VMEM((256, W), F32),
            pltpu.VMEM((16, W), F32),
            pltpu.VMEM_SHARED((RT, W), F32),
            pltpu.SemaphoreType.DMA,
        ],
    )
    def k(idx_hbm, deg_hbm, idxc, lidx_c, ones, bounce, zbuf, spm, sem):
        cid = lax.axis_index("c")
        tid = lax.axis_index("s")
        tbase = tid * S
        for i in range(SCAN):
            ones[i, pl.ds(0, 16)] = jnp.ones((16,), F32)
        for i in range(16):
            zbuf[i, pl.ds(0, 16)] = jnp.zeros((16,), F32)
        trash = jnp.full((16,), R, jnp.int32)
        for p in range(P):
            base = cid * (P * R) + p * R
            for zc in range(RZ // 16):
                pltpu.sync_copy(zbuf, spm.at[pl.ds(tid * RZ + zc * 16, 16)])
            plsc.subcore_barrier()

            def chunk(w, carry):
                pltpu.sync_copy(
                    idx_hbm.at[pl.ds(tbase + w * SCAN, SCAN)], idxc)

                def win(i, c):
                    v16 = idxc[pl.ds(i * 16, 16)]
                    mask = (v16 >= base) & (v16 < base + R)
                    lidx_c[pl.ds(i * 16, 16)] = jnp.where(mask, v16 - base,
                                                          trash)
                    return c

                lax.fori_loop(0, SCAN // 16, win, 0)
                pltpu.sync_copy(ones, spm.at[lidx_c], add=True)
                return carry

            lax.fori_loop(0, S // SCAN, chunk, 0)
            plsc.subcore_barrier()
            rbase = tid * RZ
            nfull, rem = divmod(RZ, 256)
            wchunks = [(i * 256, 256) for i in range(nfull)]
            if rem:
                wchunks.append((nfull * 256, rem))
            for oc, ln in wchunks:
                pltpu.sync_copy(spm.at[pl.ds(rbase + oc, ln)],
                                bounce.at[pl.ds(0, ln)])
                pltpu.sync_copy(bounce.at[pl.ds(0, ln)],
                                deg_hbm.at[pl.ds(base + rbase + oc, ln)])
            plsc.subcore_barrier()

    return k(idx)


def _k4_body(e_ref, W2, b2, W3, g, be, msg_ref):
    h = _leaky(e_ref[...])
    h = _leaky(_dot(h, W2[...]) + b2[...])
    msg_ref[...] = _ln(_dot(h, W3[...]), g[...], be[...])


def _run_k4(E, p, ne, grid):
    ed = p['proc'][1]['edge']
    r2 = lambda v: v.reshape(1, -1)
    args = [E, ed['W2'], r2(ed['b2']), ed['W3'], r2(ed['g']), r2(ed['be'])]
    specs = [_rows()] + [_full(a.shape) for a in args[1:]]
    return pl.pallas_call(
        _k4_body,
        grid=(grid,),
        in_specs=specs,
        out_specs=_rows(),
        out_shape=jax.ShapeDtypeStruct((ne, HID), F32),
    )(*args)


def _k6_body(h1_ref, ag_ref, nW1, nb1, nW2, nb2, nW3, ng, nbe,
             uW1, ub1, uW2, ub2, uW3, ug, ube, up_ref):
    z = _leaky(_dot(h1_ref[...], nW1[0]) + _dot(ag_ref[...], nW1[1]) + nb1[...])
    z = _leaky(_dot(z, nW2[...]) + nb2[...])
    h2 = _ln(_dot(z, nW3[...]), ng[...], nbe[...])
    t = [_bf(_leaky(h2 * uW1[j, :] + ub1[j, :])) for j in range(4)]
    u = []
    for j in range(4):
        s = t[0] * uW2[j, :]
        for k in range(1, 4):
            s = s + t[k] * uW2[k * 4 + j, :]
        u.append(_bf(_leaky(s + ub2[j, :])))
    v = []
    for j in range(4):
        s = u[0] * uW3[j, :]
        for k in range(1, 4):
            s = s + u[k] * uW3[k * 4 + j, :]
        v.append(s)
    mu = (v[0] + v[1] + v[2] + v[3]) * 0.25
    var = jnp.zeros_like(mu)
    d = []
    for j in range(4):
        dj = v[j] - mu
        d.append(dj)
        var = var + dj * dj
    inv = jax.lax.rsqrt(var * 0.25 + 1e-5)
    for j in range(4):
        up_ref[:, j, :] = d[j] * inv * ug[j, :] + ube[j, :]


def _run_k6(h1, aggr, p, np_, grid):
    nd = p['proc'][1]['node']
    du = p['dec_up']
    r2 = lambda v: v.reshape(1, -1)
    uW1 = du['W1'][:, 0, :].T
    uW2 = _bf(du['W2'].transpose(1, 2, 0).reshape(16, HID))
    uW3 = _bf(du['W3'].transpose(1, 2, 0).reshape(16, HID))
    args = [h1, aggr,
            nd['W1'].reshape(2, HID, HID), r2(nd['b1']), nd['W2'],
            r2(nd['b2']), nd['W3'], r2(nd['g']), r2(nd['be']),
            uW1, du['b1'].T, uW2, du['b2'].T, uW3, du['g'].T, du['be'].T]
    specs = [_rows(), _rows()] + [_full(a.shape) for a in args[2:]]
    return pl.pallas_call(
        _k6_body,
        grid=(grid,),
        in_specs=specs,
        out_specs=pl.BlockSpec((BLK, 4, HID), lambda i: (i, 0, 0)),
        out_shape=jax.ShapeDtypeStruct((np_, 4, HID), F32),
    )(*args)


def _k8_body(x_ref, W1, b1, W2, b2, W3, out_ref):
    h = _leaky(_dot(x_ref[...], W1[...]) + b1[...])
    h = _leaky(_dot(h, W2[...]) + b2[...])
    out_ref[...] = _dot(h, W3[...])


def _run_k8(acc, p, nn, grid):
    dc = p['dec_con']
    W1 = jnp.zeros((HID, HID), F32).at[:, :3].set(dc['W1'])
    b1 = jnp.zeros((1, HID), F32).at[0, :3].set(dc['b1'])
    W2 = jnp.zeros((HID, HID), F32).at[:3, :3].set(dc['W2'])
    b2 = jnp.zeros((1, HID), F32).at[0, :3].set(dc['b2'])
    W3 = jnp.zeros((HID, HID), F32).at[:3, :3].set(dc['W3'])
    args = [acc, W1, b1, W2, b2, W3]
    specs = [_rows()] + [_full(a.shape) for a in args[1:]]
    return pl.pallas_call(
        _k8_body,
        grid=(grid,),
        in_specs=specs,
        out_specs=_rows(),
        out_shape=jax.ShapeDtypeStruct((nn, HID), F32),
    )(*args)


def kernel(x, elem_conn, elem_index, params):
    del x
    n_elem = elem_conn.shape[0]
    src = elem_index[0]
    dest = elem_index[1]
    n_edge = dest.shape[0]
    n_nodes = 50000

    NP = ((n_elem + 1 + 25599) // 25600) * 25600
    NN = ((n_nodes + 1 + 25599) // 25600) * 25600
    EC = NW * 256
    NE = ((n_edge + EC - 1) // EC) * EC

    pad_i = jnp.full((NE - n_edge,), n_elem, jnp.int32)
    dest_p = jnp.concatenate([dest.astype(jnp.int32), pad_i])
    src_p = jnp.concatenate([src.astype(jnp.int32), pad_i])

    deg = jnp.zeros((n_elem,), F32).at[dest].add(1.0)
    deg_col = jnp.zeros((NP, 1), F32).at[:n_elem, 0].set(deg)

    berow = params['enc_conv']['be'].reshape(1, 3)
    h1, A, B = _run_k2(deg_col, berow, params, NP, NP // BLK)

    E = _sc_gather(A, B, dest_p, src_p, NE)
    msg = _run_k4(E, params, NE, NE // BLK)

    aggr = _sc_scatter_rows(msg, dest_p, NE, NP, P=NP // 12800, R=6400)

    up = _run_k6(h1, aggr, params, NP, NP // BLK)

    MN = 4 * NP
    conn_p = jnp.concatenate(
        [elem_conn.reshape(-1).astype(jnp.int32),
         jnp.full((MN - 4 * n_elem,), n_nodes, jnp.int32)])
    acc = _sc_scatter_rows(up.reshape(MN, HID), conn_p, MN, NN,
                           P=NN // 12800, R=6400)

    out = _run_k8(acc, params, NN, NN // BLK)
    return out[:n_nodes, :3]


# speedup vs baseline: 1.8519x; 1.8519x over previous
import functools

import jax
import jax.numpy as jnp
from jax import lax
from jax.experimental import pallas as pl
from jax.experimental.pallas import tpu as pltpu
from jax.experimental.pallas import tpu_sc as plsc

NC = 2
NS = 16
NW = NC * NS

F32 = jnp.float32
HID = 128
BLK = 2048


def _leaky(x):
    return jnp.where(x >= 0, x, 0.2 * x)


def _dot(a, b):
    return jax.lax.dot_general(
        a, b, (((1,), (0,)), ((), ())), preferred_element_type=F32)


def _bf(v):
    return v.astype(jnp.bfloat16).astype(F32)


def _ln(h, g, be):
    mu = jnp.mean(h, axis=-1, keepdims=True)
    d = h - mu
    var = jnp.mean(d * d, axis=-1, keepdims=True)
    return d * jax.lax.rsqrt(var + 1e-5) * g + be


def _full(shape):
    return pl.BlockSpec(shape, lambda i: tuple(0 for _ in shape))


def _rows(last=HID):
    return pl.BlockSpec((BLK, last), lambda i: (i, 0))


def _k2_body(deg_ref, berow_ref, eW1, eb1, eW2, eb2, eW3, eg, ebe,
             edW1, edb1, edW2, edb2, edW3, edg, edbe,
             nW1, nb1, nW2, nb2, nW3, ng, nbe,
             e2W1, e2b1,
             h1_ref, a_ref, b_ref):
    h = _leaky(_dot(berow_ref[...], eW1[...]) + eb1[...])
    h = _leaky(_dot(h, eW2[...]) + eb2[...])
    h0 = _ln(_dot(h, eW3[...]), eg[...], ebe[...])
    t = _leaky(_dot(h0, edW1[0]) + _dot(h0, edW1[1]) + edb1[...])
    t = _leaky(_dot(t, edW2[...]) + edb2[...])
    m = _ln(_dot(t, edW3[...]), edg[...], edbe[...])
    c0 = _dot(h0, nW1[0]) + nb1[...]
    z2 = deg_ref[...] * m
    z = _leaky(_dot(z2, nW1[1]) + c0)
    z = _leaky(_dot(z, nW2[...]) + nb2[...])
    h1 = _ln(_dot(z, nW3[...]), ng[...], nbe[...])
    h1_ref[...] = h1
    a_ref[...] = _dot(h1, e2W1[0]) + e2b1[...]
    b_ref[...] = _dot(h1, e2W1[1])


def _run_k2(deg_col, berow, p, np_, grid):
    enc = p['enc_exp']
    ed0 = p['proc'][0]['edge']
    nd0 = p['proc'][0]['node']
    ed1 = p['proc'][1]['edge']
    r2 = lambda v: v.reshape(1, -1)
    args = [
        deg_col, berow,
        enc['W1'], r2(enc['b1']), enc['W2'], r2(enc['b2']), enc['W3'],
        r2(enc['g']), r2(enc['be']),
        ed0['W1'].reshape(2, HID, HID), r2(ed0['b1']), ed0['W2'],
        r2(ed0['b2']), ed0['W3'], r2(ed0['g']), r2(ed0['be']),
        nd0['W1'].reshape(2, HID, HID), r2(nd0['b1']), nd0['W2'],
        r2(nd0['b2']), nd0['W3'], r2(nd0['g']), r2(nd0['be']),
        ed1['W1'].reshape(2, HID, HID), r2(ed1['b1']),
    ]
    specs = [_rows(1)] + [_full(a.shape) for a in args[1:]]
    out = pl.pallas_call(
        _k2_body,
        grid=(grid,),
        in_specs=specs,
        out_specs=[_rows(), _rows(), _rows()],
        out_shape=[jax.ShapeDtypeStruct((np_, HID), F32)] * 3,
    )(*args)
    return out


def _sc_gather(A, B, desti, srci, ne_pad):
    per_w = ne_pad // NW
    C = 256
    n_chunks = per_w // C
    mesh = plsc.VectorSubcoreMesh(
        core_axis_name="c", subcore_axis_name="s",
        num_cores=NC, num_subcores=NS)

    @functools.partial(
        pl.kernel,
        out_type=jax.ShapeDtypeStruct((ne_pad, HID), F32),
        mesh=mesh,
        scratch_types=[
            pltpu.VMEM((C,), jnp.int32),
            pltpu.VMEM((C,), jnp.int32),
            pltpu.VMEM((C, HID), F32),
            pltpu.SemaphoreType.DMA,
        ],
    )
    def k(dest_hbm, src_hbm, a_hbm, b_hbm, e_hbm, idxd, idxs, rows, sem):
        wid = lax.axis_index("s") * NC + lax.axis_index("c")
        base = wid * per_w

        def body(g, carry):
            off = base + g * C
            pltpu.sync_copy(dest_hbm.at[pl.ds(off, C)], idxd)
            pltpu.sync_copy(src_hbm.at[pl.ds(off, C)], idxs)
            pltpu.async_copy(a_hbm.at[idxd], rows, sem).wait()
            pltpu.async_copy(b_hbm.at[idxs], rows, sem, add=True).wait()
            pltpu.sync_copy(rows, e_hbm.at[pl.ds(off, C)])
            return carry

        lax.fori_loop(0, n_chunks, body, 0)

    return k(desti, srci, A, B)


def _k4_body(e_ref, W2, b2, W3, g, be, msg_ref):
    h = _leaky(e_ref[...])
    h = _leaky(_dot(h, W2[...]) + b2[...])
    msg_ref[...] = _ln(_dot(h, W3[...]), g[...], be[...])


def _run_k4(E, p, ne, grid):
    ed = p['proc'][1]['edge']
    r2 = lambda v: v.reshape(1, -1)
    args = [E, ed['W2'], r2(ed['b2']), ed['W3'], r2(ed['g']), r2(ed['be'])]
    specs = [_rows()] + [_full(a.shape) for a in args[1:]]
    return pl.pallas_call(
        _k4_body,
        grid=(grid,),
        in_specs=specs,
        out_specs=_rows(),
        out_shape=jax.ShapeDtypeStruct((ne, HID), F32),
    )(*args)


def _k6_body(h1_ref, ag_ref, nW1, nb1, nW2, nb2, nW3, ng, nbe,
             uW1, ub1, uW2, ub2, uW3, ug, ube, up_ref):
    z = _leaky(_dot(h1_ref[...], nW1[0]) + _dot(ag_ref[...], nW1[1]) + nb1[...])
    z = _leaky(_dot(z, nW2[...]) + nb2[...])
    h2 = _ln(_dot(z, nW3[...]), ng[...], nbe[...])
    t = [_bf(_leaky(h2 * uW1[j, :] + ub1[j, :])) for j in range(4)]
    u = []
    for j in range(4):
        s = t[0] * uW2[j, :]
        for k in range(1, 4):
            s = s + t[k] * uW2[k * 4 + j, :]
        u.append(_bf(_leaky(s + ub2[j, :])))
    v = []
    for j in range(4):
        s = u[0] * uW3[j, :]
        for k in range(1, 4):
            s = s + u[k] * uW3[k * 4 + j, :]
        v.append(s)
    mu = (v[0] + v[1] + v[2] + v[3]) * 0.25
    var = jnp.zeros_like(mu)
    d = []
    for j in range(4):
        dj = v[j] - mu
        d.append(dj)
        var = var + dj * dj
    inv = jax.lax.rsqrt(var * 0.25 + 1e-5)
    for j in range(4):
        up_ref[:, j, :] = d[j] * inv * ug[j, :] + ube[j, :]


def _run_k6(h1, aggr, p, np_, grid):
    nd = p['proc'][1]['node']
    du = p['dec_up']
    r2 = lambda v: v.reshape(1, -1)
    uW1 = du['W1'][:, 0, :].T
    uW2 = _bf(du['W2'].transpose(1, 2, 0).reshape(16, HID))
    uW3 = _bf(du['W3'].transpose(1, 2, 0).reshape(16, HID))
    args = [h1, aggr,
            nd['W1'].reshape(2, HID, HID), r2(nd['b1']), nd['W2'],
            r2(nd['b2']), nd['W3'], r2(nd['g']), r2(nd['be']),
            uW1, du['b1'].T, uW2, du['b2'].T, uW3, du['g'].T, du['be'].T]
    specs = [_rows(), _rows()] + [_full(a.shape) for a in args[2:]]
    return pl.pallas_call(
        _k6_body,
        grid=(grid,),
        in_specs=specs,
        out_specs=pl.BlockSpec((BLK, 4, HID), lambda i: (i, 0, 0)),
        out_shape=jax.ShapeDtypeStruct((np_, 4, HID), F32),
    )(*args)


def _k8_body(x_ref, W1, b1, W2, b2, W3, out_ref):
    h = _leaky(_dot(x_ref[...], W1[...]) + b1[...])
    h = _leaky(_dot(h, W2[...]) + b2[...])
    out_ref[...] = _dot(h, W3[...])


def _run_k8(acc, p, nn, grid):
    dc = p['dec_con']
    W1 = jnp.zeros((HID, HID), F32).at[:, :3].set(dc['W1'])
    b1 = jnp.zeros((1, HID), F32).at[0, :3].set(dc['b1'])
    W2 = jnp.zeros((HID, HID), F32).at[:3, :3].set(dc['W2'])
    b2 = jnp.zeros((1, HID), F32).at[0, :3].set(dc['b2'])
    W3 = jnp.zeros((HID, HID), F32).at[:3, :3].set(dc['W3'])
    args = [acc, W1, b1, W2, b2, W3]
    specs = [_rows()] + [_full(a.shape) for a in args[1:]]
    return pl.pallas_call(
        _k8_body,
        grid=(grid,),
        in_specs=specs,
        out_specs=_rows(),
        out_shape=jax.ShapeDtypeStruct((nn, HID), F32),
    )(*args)


def kernel(x, elem_conn, elem_index, params):
    del x
    n_elem = elem_conn.shape[0]
    src = elem_index[0]
    dest = elem_index[1]
    n_edge = dest.shape[0]
    n_nodes = 50000

    NP = ((n_elem + 1 + 25599) // 25600) * 25600
    NN = ((n_nodes + 1 + 25599) // 25600) * 25600
    EC = NW * 256
    NE = ((n_edge + EC - 1) // EC) * EC

    pad_i = jnp.full((NE - n_edge,), n_elem, jnp.int32)
    dest_p = jnp.concatenate([dest.astype(jnp.int32), pad_i])
    src_p = jnp.concatenate([src.astype(jnp.int32), pad_i])

    deg = jnp.zeros((n_elem,), F32).at[dest].add(1.0)
    deg_col = jnp.zeros((NP, 1), F32).at[:n_elem, 0].set(deg)

    berow = params['enc_conv']['be'].reshape(1, 3)
    h1, A, B = _run_k2(deg_col, berow, params, NP, NP // BLK)

    E = _sc_gather(A, B, dest_p, src_p, NE)
    msg = _run_k4(E, params, NE, NE // BLK)

    aggr = jnp.zeros((NP, HID), F32).at[dest].add(msg[:n_edge])

    up = _run_k6(h1, aggr, params, NP, NP // BLK)

    MN = 4 * NP
    conn_p = jnp.concatenate(
        [elem_conn.reshape(-1).astype(jnp.int32),
         jnp.full((MN - 4 * n_elem,), n_nodes, jnp.int32)])
    acc = jnp.zeros((NN, HID), F32).at[conn_p].add(up.reshape(MN, HID))

    out = _run_k8(acc, params, NN, NN // BLK)
    return out[:n_nodes, :3]
